# baseline (speedup 1.0000x reference)
import functools

import jax
import jax.numpy as jnp
from jax import lax
from jax.experimental import pallas as pl
from jax.experimental.pallas import tpu as pltpu
from jax.experimental.pallas import tpu_sc as plsc

_SPARSITY = 0.15
_GAIN = 3.0
_LEAK = 0.1

_NC = 2
_NS = 16
_L = 16
_NBINS = 256
_NVREG_HIST = _NBINS // _L


def _find_bucket(hists, kk):
    iota = lax.iota(jnp.int32, _L)
    carry = jnp.int32(0)
    cs = [None] * _NVREG_HIST
    for j in range(_NVREG_HIST - 1, -1, -1):
        h = hists[j]
        sfx = lax.rev(plsc.cumsum(lax.rev(h, (0,))), (0,))
        cs[j] = sfx + carry
        carry = carry + jnp.sum(h)
    bucket = jnp.int32(-1)
    for j in range(_NVREG_HIST):
        ib = iota + jnp.int32(j * _L)
        cand = jnp.where(cs[j] >= kk, ib, jnp.int32(-1))
        bucket = jnp.maximum(bucket, jnp.max(cand))
    c_at = jnp.int32(0)
    h_at = jnp.int32(0)
    for j in range(_NVREG_HIST):
        ib = iota + jnp.int32(j * _L)
        hit = ib == bucket
        c_at = c_at + jnp.sum(jnp.where(hit, cs[j], jnp.int32(0)))
        h_at = h_at + jnp.sum(jnp.where(hit, hists[j], jnp.int32(0)))
    n_above = c_at - h_at
    return bucket, kk - n_above


def _clear_hist(hist_ref):
    zeros = jnp.zeros((_L,), jnp.int32)
    for j in range(_NVREG_HIST):
        hist_ref[pl.ds(j * _L, _L)] = zeros


def _read_hist(hist_ref):
    return [hist_ref[pl.ds(j * _L, _L)] for j in range(_NVREG_HIST)]


def _keyize(v):
    b = lax.bitcast_convert_type(v, jnp.uint32)
    neg = (b >> jnp.uint32(31)) != jnp.uint32(0)
    return jnp.where(neg, ~b, b | jnp.uint32(0x80000000))


def kernel(x):
    rows, n = x.shape
    k = max(int(n * _SPARSITY), 1)
    nw = _NC * _NS
    rows_per_w = rows // nw
    nvec = n // _L
    mesh = plsc.VectorSubcoreMesh(core_axis_name="c", subcore_axis_name="s")

    @functools.partial(
        pl.kernel,
        out_type=jax.ShapeDtypeStruct((rows, n), jnp.float32),
        mesh=mesh,
        compiler_params=pltpu.CompilerParams(needs_layout_passes=False),
        scratch_types=[
            pltpu.VMEM((n,), jnp.float32),
            pltpu.VMEM((n,), jnp.float32),
            pltpu.VMEM((n + _L,), jnp.float32),
            pltpu.VMEM((_NBINS,), jnp.int32),
            pltpu.SemaphoreType.DMA,
            pltpu.SemaphoreType.DMA,
            pltpu.SemaphoreType.DMA,
        ],
    )
    def sc_topk_mask(x_hbm, out_hbm, xa, xb, cbuf, hist,
                     sema, semb, semo):
        wid = lax.axis_index("s") * _NC + lax.axis_index("c")
        row0 = wid * rows_per_w
        cview = cbuf.at[pl.ds(0, n)]

        def do_row(row, xbuf):
            _clear_hist(hist)

            @plsc.parallel_loop(0, nvec, unroll=8)
            def _pass0(i):
                key = _keyize(xbuf[pl.ds(i * _L, _L)])
                bucket = (key >> jnp.uint32(24)).astype(jnp.int32)
                cnt, last = plsc.scan_count(bucket)
                plsc.addupdate_scatter(hist, [bucket], cnt, mask=last)

            bucket, kk = _find_bucket(_read_hist(hist), jnp.int32(k))
            pfx8 = bucket.astype(jnp.uint32)

            pltpu.make_async_copy(cview, out_hbm.at[row], semo).wait()

            _clear_hist(hist)
            off0 = jnp.zeros((_L,), jnp.int32)

            @plsc.parallel_loop(0, nvec, unroll=8, carry=off0)
            def _pass1(i, off):
                key = _keyize(xbuf[pl.ds(i * _L, _L)])
                match = (key >> jnp.uint32(24)) == pfx8
                bucket = ((key >> jnp.uint32(16))
                          & jnp.uint32(0xFF)).astype(jnp.int32)
                cnt, last = plsc.scan_count(bucket, mask=match)
                plsc.addupdate_scatter(hist, [bucket], cnt,
                                       mask=last & match)
                csum = plsc.cumsum(match.astype(jnp.int32))
                idx = off + csum - jnp.int32(1)
                plsc.store_scatter(
                    cbuf, [idx],
                    lax.bitcast_convert_type(key, jnp.float32), mask=match)
                return off + plsc.all_reduce_population_count(match)

            off_final = _pass1
            bucket, kk = _find_bucket(_read_hist(hist), kk)
            prefix = (pfx8 << jnp.uint32(8)) | bucket.astype(jnp.uint32)

            cnt_sc = jnp.max(off_final)
            padkey = (~prefix) << jnp.uint32(16)
            plsc.store_scatter(
                cbuf, [off_final + lax.iota(jnp.int32, _L)],
                lax.bitcast_convert_type(
                    jnp.full((_L,), padkey, dtype=jnp.uint32), jnp.float32))
            ct = lax.shift_right_logical(cnt_sc + jnp.int32(15), 4)

            for lvl in (2, 3):
                shift_b = jnp.uint32(24 - 8 * lvl)
                shift_p = jnp.uint32(32 - 8 * lvl)
                _clear_hist(hist)
                pfx = prefix

                @plsc.parallel_loop(0, ct, unroll=4)
                def _passc(i, shift_b=shift_b, shift_p=shift_p, pfx=pfx):
                    key = lax.bitcast_convert_type(cbuf[pl.ds(i * _L, _L)],
                                                   jnp.uint32)
                    match = (key >> shift_p) == pfx
                    bucket = ((key >> shift_b)
                              & jnp.uint32(0xFF)).astype(jnp.int32)
                    cnt, last = plsc.scan_count(bucket, mask=match)
                    plsc.addupdate_scatter(hist, [bucket], cnt,
                                           mask=last & match)

                bucket, kk = _find_bucket(_read_hist(hist), kk)
                prefix = (prefix << jnp.uint32(8)) | bucket.astype(jnp.uint32)

            tvec = jnp.full((_L,), prefix, dtype=jnp.uint32)
            tneg = tvec < jnp.uint32(0x80000000)
            tbits = jnp.where(tneg, ~tvec, tvec ^ jnp.uint32(0x80000000))
            thr = lax.bitcast_convert_type(tbits, jnp.float32)

            @plsc.parallel_loop(0, nvec, unroll=8)
            def _passo(i):
                v = xbuf[pl.ds(i * _L, _L)]
                m = jnp.where(v >= thr, jnp.float32(1.0), jnp.float32(_LEAK))
                cbuf[pl.ds(i * _L, _L)] = v * m * jnp.float32(_GAIN)

            pltpu.async_copy(cview, out_hbm.at[row], semo)

        pltpu.async_copy(x_hbm.at[row0], xa, sema)
        pltpu.async_copy(cview, out_hbm.at[row0], semo)

        def body2(j, carry_unused):
            ra = row0 + 2 * j
            rb = ra + 1
            pltpu.async_copy(x_hbm.at[rb], xb, semb)
            pltpu.make_async_copy(x_hbm.at[ra], xa, sema).wait()
            do_row(ra, xa)
            nxt = jnp.minimum(ra + 2, jnp.int32(rows - 1))
            pltpu.async_copy(x_hbm.at[nxt], xa, sema)
            pltpu.make_async_copy(x_hbm.at[rb], xb, semb).wait()
            do_row(rb, xb)
            return carry_unused

        lax.fori_loop(0, rows_per_w // 2, body2, 0)
        pltpu.make_async_copy(x_hbm.at[row0], xa, sema).wait()
        pltpu.make_async_copy(cview, out_hbm.at[row0], semo).wait()

    return sc_topk_mask(x)


# speedup vs baseline: 1.0295x; 1.0295x over previous
import functools

import jax
import jax.numpy as jnp
from jax import lax
from jax.experimental import pallas as pl
from jax.experimental.pallas import tpu as pltpu
from jax.experimental.pallas import tpu_sc as plsc

_SPARSITY = 0.15
_GAIN = 3.0
_LEAK = 0.1

_NC = 2
_NS = 16
_L = 16
_NBINS = 256
_NVREG_HIST = _NBINS // _L


def _find_bucket(hists, kk):
    iota = lax.iota(jnp.int32, _L)
    carry = jnp.int32(0)
    cs = [None] * _NVREG_HIST
    for j in range(_NVREG_HIST - 1, -1, -1):
        h = hists[j]
        sfx = lax.rev(plsc.cumsum(lax.rev(h, (0,))), (0,))
        cs[j] = sfx + carry
        carry = carry + jnp.sum(h)
    bucket = jnp.int32(-1)
    for j in range(_NVREG_HIST):
        ib = iota + jnp.int32(j * _L)
        cand = jnp.where(cs[j] >= kk, ib, jnp.int32(-1))
        bucket = jnp.maximum(bucket, jnp.max(cand))
    c_at = jnp.int32(0)
    h_at = jnp.int32(0)
    for j in range(_NVREG_HIST):
        ib = iota + jnp.int32(j * _L)
        hit = ib == bucket
        c_at = c_at + jnp.sum(jnp.where(hit, cs[j], jnp.int32(0)))
        h_at = h_at + jnp.sum(jnp.where(hit, hists[j], jnp.int32(0)))
    n_above = c_at - h_at
    return bucket, kk - n_above


def _clear_hist(hist_ref):
    zeros = jnp.zeros((_L,), jnp.int32)
    for j in range(_NVREG_HIST):
        hist_ref[pl.ds(j * _L, _L)] = zeros


def _read_hist(hist_ref):
    return [hist_ref[pl.ds(j * _L, _L)] for j in range(_NVREG_HIST)]


def _keyize(v):
    b = lax.bitcast_convert_type(v, jnp.uint32)
    neg = (b >> jnp.uint32(31)) != jnp.uint32(0)
    return jnp.where(neg, ~b, b | jnp.uint32(0x80000000))


def kernel(x):
    rows, n = x.shape
    k = max(int(n * _SPARSITY), 1)
    nw = _NC * _NS
    rows_per_w = rows // nw
    nvec = n // _L
    mesh = plsc.VectorSubcoreMesh(core_axis_name="c", subcore_axis_name="s")

    @functools.partial(
        pl.kernel,
        out_type=jax.ShapeDtypeStruct((rows, n), jnp.float32),
        mesh=mesh,
        compiler_params=pltpu.CompilerParams(needs_layout_passes=False),
        scratch_types=[
            pltpu.VMEM((n,), jnp.float32),
            pltpu.VMEM((n,), jnp.float32),
            pltpu.VMEM((n + _L,), jnp.float32),
            pltpu.VMEM((_NBINS,), jnp.int32),
            pltpu.SemaphoreType.DMA,
            pltpu.SemaphoreType.DMA,
            pltpu.SemaphoreType.DMA,
        ],
    )
    def sc_topk_mask(x_hbm, out_hbm, xa, xb, cbuf, hist,
                     sema, semb, semo):
        wid = lax.axis_index("s") * _NC + lax.axis_index("c")
        row0 = wid * rows_per_w
        cview = cbuf.at[pl.ds(0, n)]

        def do_row(row, xbuf):
            _clear_hist(hist)

            @plsc.parallel_loop(0, nvec, unroll=8)
            def _pass0(i):
                key = _keyize(xbuf[pl.ds(i * _L, _L)])
                bucket = (key >> jnp.uint32(24)).astype(jnp.int32)
                cnt, last = plsc.scan_count(bucket)
                plsc.addupdate_scatter(hist, [bucket], cnt, mask=last)

            bucket, kk = _find_bucket(_read_hist(hist), jnp.int32(k))
            pfx8 = bucket.astype(jnp.uint32)

            pltpu.make_async_copy(cview, out_hbm.at[row], semo).wait()

            off0 = jnp.zeros((_L,), jnp.int32)

            @plsc.parallel_loop(0, nvec, unroll=8, carry=off0)
            def _pass1(i, off):
                key = _keyize(xbuf[pl.ds(i * _L, _L)])
                match = (key >> jnp.uint32(24)) == pfx8
                csum = plsc.cumsum(match.astype(jnp.int32))
                idx = off + csum - jnp.int32(1)
                plsc.store_scatter(
                    cbuf, [idx],
                    lax.bitcast_convert_type(key, jnp.float32), mask=match)
                return off + plsc.all_reduce_population_count(match)

            off_final = _pass1
            prefix = pfx8

            cnt_sc = jnp.max(off_final)
            padkey = (~pfx8) << jnp.uint32(24)
            plsc.store_scatter(
                cbuf, [off_final + lax.iota(jnp.int32, _L)],
                lax.bitcast_convert_type(
                    jnp.full((_L,), padkey, dtype=jnp.uint32), jnp.float32))
            ct = lax.shift_right_logical(cnt_sc + jnp.int32(15), 4)

            for lvl in (1, 2, 3):
                shift_b = jnp.uint32(24 - 8 * lvl)
                shift_p = jnp.uint32(32 - 8 * lvl)
                _clear_hist(hist)
                pfx = prefix

                @plsc.parallel_loop(0, ct, unroll=4)
                def _passc(i, shift_b=shift_b, shift_p=shift_p, pfx=pfx):
                    key = lax.bitcast_convert_type(cbuf[pl.ds(i * _L, _L)],
                                                   jnp.uint32)
                    match = (key >> shift_p) == pfx
                    bucket = ((key >> shift_b)
                              & jnp.uint32(0xFF)).astype(jnp.int32)
                    cnt, last = plsc.scan_count(bucket, mask=match)
                    plsc.addupdate_scatter(hist, [bucket], cnt,
                                           mask=last & match)

                bucket, kk = _find_bucket(_read_hist(hist), kk)
                prefix = (prefix << jnp.uint32(8)) | bucket.astype(jnp.uint32)

            tvec = jnp.full((_L,), prefix, dtype=jnp.uint32)
            tneg = tvec < jnp.uint32(0x80000000)
            tbits = jnp.where(tneg, ~tvec, tvec ^ jnp.uint32(0x80000000))
            thr = lax.bitcast_convert_type(tbits, jnp.float32)

            @plsc.parallel_loop(0, nvec, unroll=8)
            def _passo(i):
                v = xbuf[pl.ds(i * _L, _L)]
                m = jnp.where(v >= thr, jnp.float32(1.0), jnp.float32(_LEAK))
                cbuf[pl.ds(i * _L, _L)] = v * m * jnp.float32(_GAIN)

            pltpu.async_copy(cview, out_hbm.at[row], semo)

        pltpu.async_copy(x_hbm.at[row0], xa, sema)
        pltpu.async_copy(cview, out_hbm.at[row0], semo)

        def body2(j, carry_unused):
            ra = row0 + 2 * j
            rb = ra + 1
            pltpu.async_copy(x_hbm.at[rb], xb, semb)
            pltpu.make_async_copy(x_hbm.at[ra], xa, sema).wait()
            do_row(ra, xa)
            nxt = jnp.minimum(ra + 2, jnp.int32(rows - 1))
            pltpu.async_copy(x_hbm.at[nxt], xa, sema)
            pltpu.make_async_copy(x_hbm.at[rb], xb, semb).wait()
            do_row(rb, xb)
            return carry_unused

        lax.fori_loop(0, rows_per_w // 2, body2, 0)
        pltpu.make_async_copy(x_hbm.at[row0], xa, sema).wait()
        pltpu.make_async_copy(cview, out_hbm.at[row0], semo).wait()

    return sc_topk_mask(x)


# speedup vs baseline: 1.0889x; 1.0577x over previous
import functools

import jax
import jax.numpy as jnp
from jax import lax
from jax.experimental import pallas as pl
from jax.experimental.pallas import tpu as pltpu
from jax.experimental.pallas import tpu_sc as plsc

_SPARSITY = 0.15
_GAIN = 3.0
_LEAK = 0.1

_NC = 2
_NS = 16
_L = 16
_NB0 = 16384
_NGRP = 64
_NB12 = 512


def _find_bucket(hists, kk):
    nv = len(hists)
    iota = lax.iota(jnp.int32, _L)
    carry = jnp.int32(0)
    cs = [None] * nv
    for j in range(nv - 1, -1, -1):
        h = hists[j]
        sfx = lax.rev(plsc.cumsum(lax.rev(h, (0,))), (0,))
        cs[j] = sfx + carry
        carry = carry + jnp.sum(h)
    bucket = jnp.int32(-1)
    for j in range(nv):
        ib = iota + jnp.int32(j * _L)
        cand = jnp.where(cs[j] >= kk, ib, jnp.int32(-1))
        bucket = jnp.maximum(bucket, jnp.max(cand))
    c_at = jnp.int32(0)
    h_at = jnp.int32(0)
    for j in range(nv):
        ib = iota + jnp.int32(j * _L)
        hit = ib == bucket
        c_at = c_at + jnp.sum(jnp.where(hit, cs[j], jnp.int32(0)))
        h_at = h_at + jnp.sum(jnp.where(hit, hists[j], jnp.int32(0)))
    n_above = c_at - h_at
    return bucket, kk - n_above


def _keyize(v):
    b = lax.bitcast_convert_type(v, jnp.uint32)
    neg = (b >> jnp.uint32(31)) != jnp.uint32(0)
    return jnp.where(neg, ~b, b | jnp.uint32(0x80000000))


def kernel(x):
    rows, n = x.shape
    k = max(int(n * _SPARSITY), 1)
    nw = _NC * _NS
    rows_per_w = rows // nw
    nvec = n // _L
    mesh = plsc.VectorSubcoreMesh(core_axis_name="c", subcore_axis_name="s")

    @functools.partial(
        pl.kernel,
        out_type=jax.ShapeDtypeStruct((rows, n), jnp.float32),
        mesh=mesh,
        compiler_params=pltpu.CompilerParams(needs_layout_passes=False),
        scratch_types=[
            pltpu.VMEM((n,), jnp.float32),
            pltpu.VMEM((n,), jnp.float32),
            pltpu.VMEM((n + _L,), jnp.float32),
            pltpu.VMEM((_NB0,), jnp.int32),
            pltpu.VMEM((_NGRP * _L,), jnp.int32),
            pltpu.SemaphoreType.DMA,
            pltpu.SemaphoreType.DMA,
            pltpu.SemaphoreType.DMA,
        ],
    )
    def sc_topk_mask(x_hbm, out_hbm, xa, xb, cbuf, hist, gsum,
                     sema, semb, semo):
        wid = lax.axis_index("s") * _NC + lax.axis_index("c")
        row0 = wid * rows_per_w
        cview = cbuf.at[pl.ds(0, n)]
        zeros = jnp.zeros((_L,), jnp.int32)

        def do_row(row, xbuf):
            @plsc.parallel_loop(0, _NB0 // _L, unroll=8)
            def _clr(i):
                hist[pl.ds(i * _L, _L)] = zeros

            @plsc.parallel_loop(0, nvec, unroll=8)
            def _pass0(i):
                key = _keyize(xbuf[pl.ds(i * _L, _L)])
                bucket = (key >> jnp.uint32(18)).astype(jnp.int32)
                cnt, last = plsc.scan_count(bucket)
                plsc.addupdate_scatter(hist, [bucket], cnt, mask=last)

            def gbody(g, c):
                acc = hist[pl.ds(g * 256, _L)]
                for j in range(1, 16):
                    acc = acc + hist[pl.ds(g * 256 + j * _L, _L)]
                gsum[pl.ds(g * _L, _L)] = acc
                return c

            lax.fori_loop(0, _NGRP, gbody, 0)

            tg = [jnp.sum(gsum[pl.ds(g * _L, _L)]) for g in range(_NGRP)]
            sg = [None] * _NGRP
            run = jnp.int32(0)
            for g in range(_NGRP - 1, -1, -1):
                sg[g] = run
                run = run + tg[g]
            kk = jnp.int32(k)
            gstar = jnp.int32(-1)
            for g in range(_NGRP):
                inb = (sg[g] + tg[g]) >= kk
                gstar = jnp.maximum(gstar,
                                    jnp.where(inb, jnp.int32(g),
                                              jnp.int32(-1)))
            above = jnp.int32(0)
            for g in range(_NGRP):
                above = above + jnp.where(gstar == g, sg[g], jnp.int32(0))
            kk = kk - above
            gbase = gstar * jnp.int32(256)
            hs = [hist[pl.ds(gbase + j * _L, _L)] for j in range(16)]
            bucket, kk = _find_bucket(hs, kk)
            pfx14 = (gbase + bucket).astype(jnp.uint32)

            pltpu.make_async_copy(cview, out_hbm.at[row], semo).wait()

            pos = pfx14 >= jnp.uint32(0x2000)
            craw = jnp.where(pos, pfx14 - jnp.uint32(0x2000),
                             jnp.uint32(0x3FFF) - pfx14)
            off0 = jnp.zeros((_L,), jnp.int32)

            @plsc.parallel_loop(0, nvec, unroll=8, carry=off0)
            def _pass1(i, off):
                v = xbuf[pl.ds(i * _L, _L)]
                b = lax.bitcast_convert_type(v, jnp.uint32)
                match = (b >> jnp.uint32(18)) == craw
                csum = plsc.cumsum(match.astype(jnp.int32))
                idx = off + csum - jnp.int32(1)
                plsc.store_scatter(cbuf, [idx], v, mask=match)
                return off + plsc.all_reduce_population_count(match)

            off_final = _pass1

            cnt_sc = jnp.max(off_final)
            padkey = (~pfx14) << jnp.uint32(18)
            padbits = jnp.where(padkey < jnp.uint32(0x80000000),
                                ~padkey, padkey ^ jnp.uint32(0x80000000))
            plsc.store_scatter(
                cbuf, [off_final + lax.iota(jnp.int32, _L)],
                lax.bitcast_convert_type(
                    jnp.full((_L,), padbits, dtype=jnp.uint32),
                    jnp.float32))
            ct = lax.shift_right_logical(cnt_sc + jnp.int32(15), 4)

            prefix = pfx14
            for lvl in range(2):
                shift_b = jnp.uint32(9 - 9 * lvl)
                shift_p = jnp.uint32(18 - 9 * lvl)
                for j in range(_NB12 // _L):
                    hist[pl.ds(j * _L, _L)] = zeros
                pfx = prefix

                @plsc.parallel_loop(0, ct, unroll=4)
                def _passc(i, shift_b=shift_b, shift_p=shift_p, pfx=pfx):
                    key = _keyize(cbuf[pl.ds(i * _L, _L)])
                    match = (key >> shift_p) == pfx
                    bucket = ((key >> shift_b)
                              & jnp.uint32(0x1FF)).astype(jnp.int32)
                    cnt, last = plsc.scan_count(bucket, mask=match)
                    plsc.addupdate_scatter(hist, [bucket], cnt,
                                           mask=last & match)

                hs = [hist[pl.ds(j * _L, _L)] for j in range(_NB12 // _L)]
                bucket, kk = _find_bucket(hs, kk)
                prefix = (prefix << jnp.uint32(9)) | bucket.astype(jnp.uint32)

            tvec = jnp.full((_L,), prefix, dtype=jnp.uint32)
            tneg = tvec < jnp.uint32(0x80000000)
            tbits = jnp.where(tneg, ~tvec, tvec ^ jnp.uint32(0x80000000))
            thr = lax.bitcast_convert_type(tbits, jnp.float32)

            @plsc.parallel_loop(0, nvec, unroll=8)
            def _passo(i):
                v = xbuf[pl.ds(i * _L, _L)]
                m = jnp.where(v >= thr, jnp.float32(1.0), jnp.float32(_LEAK))
                cbuf[pl.ds(i * _L, _L)] = v * m * jnp.float32(_GAIN)

            pltpu.async_copy(cview, out_hbm.at[row], semo)

        pltpu.async_copy(x_hbm.at[row0], xa, sema)
        pltpu.async_copy(cview, out_hbm.at[row0], semo)

        def body2(j, carry_unused):
            ra = row0 + 2 * j
            rb = ra + 1
            pltpu.async_copy(x_hbm.at[rb], xb, semb)
            pltpu.make_async_copy(x_hbm.at[ra], xa, sema).wait()
            do_row(ra, xa)
            nxt = jnp.minimum(ra + 2, jnp.int32(rows - 1))
            pltpu.async_copy(x_hbm.at[nxt], xa, sema)
            pltpu.make_async_copy(x_hbm.at[rb], xb, semb).wait()
            do_row(rb, xb)
            return carry_unused

        lax.fori_loop(0, rows_per_w // 2, body2, 0)
        pltpu.make_async_copy(x_hbm.at[row0], xa, sema).wait()
        pltpu.make_async_copy(cview, out_hbm.at[row0], semo).wait()

    return sc_topk_mask(x)


# speedup vs baseline: 1.1091x; 1.0186x over previous
import functools

import jax
import jax.numpy as jnp
from jax import lax
from jax.experimental import pallas as pl
from jax.experimental.pallas import tpu as pltpu
from jax.experimental.pallas import tpu_sc as plsc

_SPARSITY = 0.15
_GAIN = 3.0
_LEAK = 0.1

_NC = 2
_NS = 16
_L = 16
_NB0 = 16384
_NGRP = 64
_NB12 = 512


def _find_bucket(hists, kk):
    nv = len(hists)
    iota = lax.iota(jnp.int32, _L)
    carry = jnp.int32(0)
    cs = [None] * nv
    for j in range(nv - 1, -1, -1):
        h = hists[j]
        sfx = lax.rev(plsc.cumsum(lax.rev(h, (0,))), (0,))
        cs[j] = sfx + carry
        carry = carry + jnp.sum(h)
    bucket = jnp.int32(-1)
    for j in range(nv):
        ib = iota + jnp.int32(j * _L)
        cand = jnp.where(cs[j] >= kk, ib, jnp.int32(-1))
        bucket = jnp.maximum(bucket, jnp.max(cand))
    c_at = jnp.int32(0)
    h_at = jnp.int32(0)
    for j in range(nv):
        ib = iota + jnp.int32(j * _L)
        hit = ib == bucket
        c_at = c_at + jnp.sum(jnp.where(hit, cs[j], jnp.int32(0)))
        h_at = h_at + jnp.sum(jnp.where(hit, hists[j], jnp.int32(0)))
    n_above = c_at - h_at
    return bucket, kk - n_above


def _keyize(v):
    b = lax.bitcast_convert_type(v, jnp.uint32)
    neg = (b >> jnp.uint32(31)) != jnp.uint32(0)
    return jnp.where(neg, ~b, b | jnp.uint32(0x80000000))


def kernel(x):
    rows, n = x.shape
    k = max(int(n * _SPARSITY), 1)
    nw = _NC * _NS
    rows_per_w = rows // nw
    nvec = n // _L
    mesh = plsc.VectorSubcoreMesh(core_axis_name="c", subcore_axis_name="s")

    @functools.partial(
        pl.kernel,
        out_type=jax.ShapeDtypeStruct((rows, n), jnp.float32),
        mesh=mesh,
        compiler_params=pltpu.CompilerParams(needs_layout_passes=False,
                                             skip_device_barrier=True),
        scratch_types=[
            pltpu.VMEM((n,), jnp.float32),
            pltpu.VMEM((n,), jnp.float32),
            pltpu.VMEM((n + _L,), jnp.float32),
            pltpu.VMEM((_NB0,), jnp.int32),
            pltpu.VMEM((_NGRP * _L,), jnp.int32),
            pltpu.SemaphoreType.DMA,
            pltpu.SemaphoreType.DMA,
            pltpu.SemaphoreType.DMA,
        ],
    )
    def sc_topk_mask(x_hbm, out_hbm, xa, xb, cbuf, hist, gsum,
                     sema, semb, semo):
        wid = lax.axis_index("s") * _NC + lax.axis_index("c")
        row0 = wid * rows_per_w
        cview = cbuf.at[pl.ds(0, n)]
        zeros = jnp.zeros((_L,), jnp.int32)

        def do_row(row, xbuf):
            @plsc.parallel_loop(0, _NB0 // _L, unroll=8)
            def _clr(i):
                hist[pl.ds(i * _L, _L)] = zeros

            @plsc.parallel_loop(0, nvec, unroll=8)
            def _pass0(i):
                key = _keyize(xbuf[pl.ds(i * _L, _L)])
                bucket = (key >> jnp.uint32(18)).astype(jnp.int32)
                cnt, last = plsc.scan_count(bucket)
                plsc.addupdate_scatter(hist, [bucket], cnt, mask=last)

            def gbody(g, c):
                acc = hist[pl.ds(g * 256, _L)]
                for j in range(1, 16):
                    acc = acc + hist[pl.ds(g * 256 + j * _L, _L)]
                gsum[pl.ds(g * _L, _L)] = acc
                return c

            lax.fori_loop(0, _NGRP, gbody, 0)

            tg = [jnp.sum(gsum[pl.ds(g * _L, _L)]) for g in range(_NGRP)]
            sg = [None] * _NGRP
            run = jnp.int32(0)
            for g in range(_NGRP - 1, -1, -1):
                sg[g] = run
                run = run + tg[g]
            kk = jnp.int32(k)
            gstar = jnp.int32(-1)
            for g in range(_NGRP):
                inb = (sg[g] + tg[g]) >= kk
                gstar = jnp.maximum(gstar,
                                    jnp.where(inb, jnp.int32(g),
                                              jnp.int32(-1)))
            above = jnp.int32(0)
            for g in range(_NGRP):
                above = above + jnp.where(gstar == g, sg[g], jnp.int32(0))
            kk = kk - above
            gbase = gstar * jnp.int32(256)
            hs = [hist[pl.ds(gbase + j * _L, _L)] for j in range(16)]
            bucket, kk = _find_bucket(hs, kk)
            pfx14 = (gbase + bucket).astype(jnp.uint32)

            pltpu.make_async_copy(cview, out_hbm.at[row], semo).wait()

            pos = pfx14 >= jnp.uint32(0x2000)
            craw = jnp.where(pos, pfx14 - jnp.uint32(0x2000),
                             jnp.uint32(0x3FFF) - pfx14)
            off0 = jnp.zeros((_L,), jnp.int32)

            @plsc.parallel_loop(0, nvec, unroll=8, carry=off0)
            def _pass1(i, off):
                v = xbuf[pl.ds(i * _L, _L)]
                b = lax.bitcast_convert_type(v, jnp.uint32)
                match = (b >> jnp.uint32(18)) == craw
                csum = plsc.cumsum(match.astype(jnp.int32))
                idx = off + csum - jnp.int32(1)
                plsc.store_scatter(cbuf, [idx], v, mask=match)
                return off + plsc.all_reduce_population_count(match)

            off_final = _pass1

            cnt_sc = jnp.max(off_final)
            padkey = (~pfx14) << jnp.uint32(18)
            padbits = jnp.where(padkey < jnp.uint32(0x80000000),
                                ~padkey, padkey ^ jnp.uint32(0x80000000))
            plsc.store_scatter(
                cbuf, [off_final + lax.iota(jnp.int32, _L)],
                lax.bitcast_convert_type(
                    jnp.full((_L,), padbits, dtype=jnp.uint32),
                    jnp.float32))
            ct = lax.shift_right_logical(cnt_sc + jnp.int32(15), 4)

            prefix = pfx14
            for lvl in range(2):
                shift_b = jnp.uint32(9 - 9 * lvl)
                shift_p = jnp.uint32(18 - 9 * lvl)
                for j in range(_NB12 // _L):
                    hist[pl.ds(j * _L, _L)] = zeros
                pfx = prefix

                @plsc.parallel_loop(0, ct, unroll=4)
                def _passc(i, shift_b=shift_b, shift_p=shift_p, pfx=pfx):
                    key = _keyize(cbuf[pl.ds(i * _L, _L)])
                    match = (key >> shift_p) == pfx
                    bucket = ((key >> shift_b)
                              & jnp.uint32(0x1FF)).astype(jnp.int32)
                    cnt, last = plsc.scan_count(bucket, mask=match)
                    plsc.addupdate_scatter(hist, [bucket], cnt,
                                           mask=last & match)

                hs = [hist[pl.ds(j * _L, _L)] for j in range(_NB12 // _L)]
                bucket, kk = _find_bucket(hs, kk)
                prefix = (prefix << jnp.uint32(9)) | bucket.astype(jnp.uint32)

            tvec = jnp.full((_L,), prefix, dtype=jnp.uint32)
            tneg = tvec < jnp.uint32(0x80000000)
            tbits = jnp.where(tneg, ~tvec, tvec ^ jnp.uint32(0x80000000))
            thr = lax.bitcast_convert_type(tbits, jnp.float32)

            @plsc.parallel_loop(0, nvec, unroll=16)
            def _passo(i):
                v = xbuf[pl.ds(i * _L, _L)]
                g = jnp.where(v >= thr, jnp.float32(_GAIN),
                              jnp.float32(_GAIN * _LEAK))
                cbuf[pl.ds(i * _L, _L)] = v * g

            pltpu.async_copy(cview, out_hbm.at[row], semo)

        pltpu.async_copy(x_hbm.at[row0], xa, sema)
        pltpu.async_copy(cview, out_hbm.at[row0], semo)

        def body2(j, carry_unused):
            ra = row0 + 2 * j
            rb = ra + 1
            pltpu.async_copy(x_hbm.at[rb], xb, semb)
            pltpu.make_async_copy(x_hbm.at[ra], xa, sema).wait()
            do_row(ra, xa)
            nxt = jnp.minimum(ra + 2, jnp.int32(rows - 1))
            pltpu.async_copy(x_hbm.at[nxt], xa, sema)
            pltpu.make_async_copy(x_hbm.at[rb], xb, semb).wait()
            do_row(rb, xb)
            return carry_unused

        lax.fori_loop(0, rows_per_w // 2, body2, 0)
        pltpu.make_async_copy(x_hbm.at[row0], xa, sema).wait()
        pltpu.make_async_copy(cview, out_hbm.at[row0], semo).wait()

    return sc_topk_mask(x)
